# Initial kernel scaffold; baseline (speedup 1.0000x reference)
import jax
import jax.numpy as jnp
from jax.experimental import pallas as pl


def kernel(l_size, c_size, l_edge_index, c_edge_index, l_emb, c_emb, W1_l2c, b1_l2c, W2_l2c, b2_l2c, W1_c2l, b1_c2l, W2_c2l, b2_c2l, Wih_c, Whh_c, bih_c, bhh_c, Wih_l, Whh_l, bih_l, bhh_l):
    raise NotImplementedError("write your pallas kernel here")


# speedup vs baseline: 2.3260x; 2.3260x over previous
import functools

import jax
import jax.numpy as jnp
from jax import lax
from jax.experimental import pallas as pl
from jax.experimental.pallas import tpu as pltpu
from jax.experimental.pallas import tpu_sc as plsc

DIM = 128
N_ITER = 4

_NCORES = 2
_NTILES = 16
_CHUNK = 128


def _sc_scatter_body(n_nodes, rows_per_tile, featL, featC, g0, s0, g1, s1,
                     out0, out1, acc, gidx_v, sidx_v, rows_v, sem):
    cid = lax.axis_index("c")
    sid = lax.axis_index("s")

    nz = (n_nodes // _NTILES) // 8 * 8
    last_extra = n_nodes - nz * _NTILES
    base = sid * nz

    def _zero_rows(i, _):
        r = i // (DIM // 16)
        c = lax.rem(i, DIM // 16)
        rows_v[r, pl.ds(c * 16, 16)] = jnp.zeros((16,), jnp.float32)
        return 0

    lax.fori_loop(0, _CHUNK * (DIM // 16), _zero_rows, 0)

    n_full = nz // _CHUNK
    rem = nz - n_full * _CHUNK
    for k in range(n_full):
        pltpu.sync_copy(rows_v, acc.at[pl.ds(base + k * _CHUNK, _CHUNK)])
    if rem:
        pltpu.sync_copy(rows_v.at[pl.ds(0, rem)],
                        acc.at[pl.ds(base + n_full * _CHUNK, rem)])

    @pl.when(sid == _NTILES - 1)
    def _():
        pltpu.sync_copy(rows_v.at[pl.ds(0, last_extra + 8)],
                        acc.at[pl.ds(nz * _NTILES, last_extra + 8)])

    plsc.subcore_barrier()

    pass_rows = gidx_v.shape[0]

    def run_dir(feat_hbm, gidx_hbm, sidx_hbm, out_hbm):
        def body(j, _):
            pltpu.async_copy(feat_hbm.at[gidx_v.at[j]], rows_v, sem).wait()
            pltpu.sync_copy(rows_v, acc.at[sidx_v.at[j]], add=True)
            return 0

        for h in range(rows_per_tile // pass_rows):
            off = sid * rows_per_tile + h * pass_rows
            pltpu.sync_copy(gidx_hbm.at[pl.ds(off, pass_rows)], gidx_v)
            pltpu.sync_copy(sidx_hbm.at[pl.ds(off, pass_rows)], sidx_v)
            lax.fori_loop(0, pass_rows, body, 0)
        plsc.subcore_barrier()

        for k in range(n_full):
            pltpu.sync_copy(acc.at[pl.ds(base + k * _CHUNK, _CHUNK)],
                            out_hbm.at[pl.ds(base + k * _CHUNK, _CHUNK)])
        if rem:
            pltpu.sync_copy(acc.at[pl.ds(base + n_full * _CHUNK, rem)],
                            out_hbm.at[pl.ds(base + n_full * _CHUNK, rem)])

        if last_extra:
            @pl.when(sid == _NTILES - 1)
            def _():
                pltpu.sync_copy(acc.at[pl.ds(nz * _NTILES, last_extra)],
                                out_hbm.at[pl.ds(nz * _NTILES, last_extra)])

    @pl.when(cid == 0)
    def _():
        run_dir(featL, g0, s0, out0)

    @pl.when(cid == 1)
    def _():
        run_dir(featC, g1, s1, out1)


@functools.lru_cache(maxsize=None)
def _make_sc_scatter(n_nodes, idx_rows):
    rows_per_tile = idx_rows // _NTILES
    pass_rows = 8
    for cand in (40, 32, 24, 16):
        if rows_per_tile % cand == 0:
            pass_rows = cand
            break
    mesh = plsc.VectorSubcoreMesh(core_axis_name="c", subcore_axis_name="s")
    f32 = jnp.float32
    return pl.kernel(
        functools.partial(_sc_scatter_body, n_nodes, rows_per_tile),
        out_type=[jax.ShapeDtypeStruct((n_nodes, DIM), f32),
                  jax.ShapeDtypeStruct((n_nodes, DIM), f32)],
        mesh=mesh,
        scratch_types=[
            pltpu.VMEM_SHARED((n_nodes + 8, DIM), f32),
            pltpu.VMEM((pass_rows, _CHUNK), jnp.int32),
            pltpu.VMEM((pass_rows, _CHUNK), jnp.int32),
            pltpu.VMEM((_CHUNK, DIM), f32),
            pltpu.SemaphoreType.DMA,
        ],
    )


_BLK = 1000


def _mlp_body(x_ref, w1_ref, b1_ref, w2_ref, b2_ref, o_ref):
    x = x_ref[...]
    h = jnp.maximum(
        jnp.dot(x, w1_ref[...], preferred_element_type=jnp.float32)
        + b1_ref[...], 0.0)
    o_ref[...] = (jnp.dot(h, w2_ref[...], preferred_element_type=jnp.float32)
                  + b2_ref[...])


def _mlp(x, w1, b1, w2, b2):
    n = x.shape[0]
    grid = n // _BLK
    return pl.pallas_call(
        _mlp_body,
        grid=(grid,),
        in_specs=[
            pl.BlockSpec((_BLK, DIM), lambda i: (i, 0)),
            pl.BlockSpec((DIM, DIM), lambda i: (0, 0)),
            pl.BlockSpec((1, DIM), lambda i: (0, 0)),
            pl.BlockSpec((DIM, DIM), lambda i: (0, 0)),
            pl.BlockSpec((1, DIM), lambda i: (0, 0)),
        ],
        out_specs=pl.BlockSpec((_BLK, DIM), lambda i: (i, 0)),
        out_shape=jax.ShapeDtypeStruct((n, DIM), jnp.float32),
    )(x, w1, b1.reshape(1, DIM), w2, b2.reshape(1, DIM))


def _gru_gates(gi, gh, h):
    r = jax.nn.sigmoid(gi[:, :DIM] + gh[:, :DIM])
    z = jax.nn.sigmoid(gi[:, DIM:2 * DIM] + gh[:, DIM:2 * DIM])
    n = jnp.tanh(gi[:, 2 * DIM:] + r * gh[:, 2 * DIM:])
    return (1.0 - z) * n + z * h


def _gru_c_body(x_ref, h_ref, wih_t, whh_t, bih, bhh, o_ref):
    x = x_ref[...]
    h = h_ref[...]
    gi = jnp.dot(x, wih_t[...], preferred_element_type=jnp.float32) + bih[...]
    gh = jnp.dot(h, whh_t[...], preferred_element_type=jnp.float32) + bhh[...]
    o_ref[...] = _gru_gates(gi, gh, h)


def _gru_c(x, h, wih_t, whh_t, bih, bhh):
    n = x.shape[0]
    grid = n // _BLK
    full = pl.BlockSpec((_BLK, DIM), lambda i: (i, 0))
    return pl.pallas_call(
        _gru_c_body,
        grid=(grid,),
        in_specs=[
            full, full,
            pl.BlockSpec((DIM, 3 * DIM), lambda i: (0, 0)),
            pl.BlockSpec((DIM, 3 * DIM), lambda i: (0, 0)),
            pl.BlockSpec((1, 3 * DIM), lambda i: (0, 0)),
            pl.BlockSpec((1, 3 * DIM), lambda i: (0, 0)),
        ],
        out_specs=full,
        out_shape=jax.ShapeDtypeStruct((n, DIM), jnp.float32),
    )(x, h, wih_t, whh_t, bih.reshape(1, -1), bhh.reshape(1, -1))


def _gru_l_body(a_ref, h_ref, wa_t, wb_t, whh_t, bih, bhh, o_ref):
    a = a_ref[...]
    h = h_ref[...]
    up = jnp.roll(h, -1, axis=0)
    down = jnp.roll(h, 1, axis=0)
    parity = lax.rem(lax.broadcasted_iota(jnp.int32, h.shape, 0), 2)
    swapped = jnp.where(parity == 0, up, down)
    gi = (jnp.dot(a, wa_t[...], preferred_element_type=jnp.float32)
          + jnp.dot(swapped, wb_t[...], preferred_element_type=jnp.float32)
          + bih[...])
    gh = jnp.dot(h, whh_t[...], preferred_element_type=jnp.float32) + bhh[...]
    o_ref[...] = _gru_gates(gi, gh, h)


def _gru_l(a, h, wa_t, wb_t, whh_t, bih, bhh):
    n = a.shape[0]
    grid = n // _BLK
    full = pl.BlockSpec((_BLK, DIM), lambda i: (i, 0))
    w = pl.BlockSpec((DIM, 3 * DIM), lambda i: (0, 0))
    return pl.pallas_call(
        _gru_l_body,
        grid=(grid,),
        in_specs=[
            full, full, w, w, w,
            pl.BlockSpec((1, 3 * DIM), lambda i: (0, 0)),
            pl.BlockSpec((1, 3 * DIM), lambda i: (0, 0)),
        ],
        out_specs=full,
        out_shape=jax.ShapeDtypeStruct((n, DIM), jnp.float32),
    )(a, h, wa_t, wb_t, whh_t, bih.reshape(1, -1), bhh.reshape(1, -1))


def kernel(l_size, c_size, l_edge_index, c_edge_index, l_emb, c_emb,
           W1_l2c, b1_l2c, W2_l2c, b2_l2c,
           W1_c2l, b1_c2l, W2_c2l, b2_c2l,
           Wih_c, Whh_c, bih_c, bhh_c,
           Wih_l, Whh_l, bih_l, bhh_l):
    ls = l_emb.shape[0]
    cs = c_emb.shape[0]
    n_nodes = ls
    e = l_edge_index.shape[0]

    per_tile = -(-e // (_NTILES * _CHUNK))
    per_tile = -(-per_tile // 8) * 8
    e_pad = per_tile * _NTILES * _CHUNK
    trash = jnp.int32(n_nodes)
    ei_l = l_edge_index.astype(jnp.int32)
    ei_c = c_edge_index.astype(jnp.int32)
    pad0 = jnp.zeros((e_pad - e,), jnp.int32)
    padt = jnp.full((e_pad - e,), trash, jnp.int32)
    rows = e_pad // _CHUNK
    g0 = jnp.concatenate([ei_l, pad0]).reshape(rows, _CHUNK)
    s0 = jnp.concatenate([ei_c, padt]).reshape(rows, _CHUNK)
    g1 = jnp.concatenate([ei_c, pad0]).reshape(rows, _CHUNK)
    s1 = jnp.concatenate([ei_l, padt]).reshape(rows, _CHUNK)

    sc_scatter = _make_sc_scatter(n_nodes, rows)

    wih_c_t = Wih_c.T
    whh_c_t = Whh_c.T
    wih_l_a = Wih_l[:, :DIM].T
    wih_l_b = Wih_l[:, DIM:].T
    whh_l_t = Whh_l.T

    l_embs = [l_emb]
    c_embs = [c_emb]
    for _ in range(N_ITER):
        featL = _mlp(l_emb, W1_l2c, b1_l2c, W2_l2c, b2_l2c)
        featC = _mlp(c_emb, W1_c2l, b1_c2l, W2_c2l, b2_c2l)
        l2c_aggr, c2l_aggr = sc_scatter(featL, featC, g0, s0, g1, s1)
        c_emb = _gru_c(l2c_aggr, c_emb, wih_c_t, whh_c_t, bih_c, bhh_c)
        l_emb = _gru_l(c2l_aggr, l_emb, wih_l_a, wih_l_b, whh_l_t,
                       bih_l, bhh_l)
        l_embs.append(l_emb)
        c_embs.append(c_emb)
    return (jnp.stack(l_embs), jnp.stack(c_embs))


# speedup vs baseline: 2.5389x; 1.0916x over previous
import functools

import jax
import jax.numpy as jnp
from jax import lax
from jax.experimental import pallas as pl
from jax.experimental.pallas import tpu as pltpu
from jax.experimental.pallas import tpu_sc as plsc

DIM = 128
N_ITER = 4

_NCORES = 2
_NTILES = 16
_CHUNK = 128


def _sc_scatter_body(n_nodes, rows_per_tile, featL, featC, g0, s0, g1, s1,
                     out0, out1, acc, gidx_v, sidx_v, rows_v, rows_w,
                     sem0, sem1):
    cid = lax.axis_index("c")
    sid = lax.axis_index("s")

    nz = (n_nodes // _NTILES) // 8 * 8
    last_extra = n_nodes - nz * _NTILES
    base = sid * nz

    def _zero_rows(i, _):
        r = i // (DIM // 16)
        c = lax.rem(i, DIM // 16)
        rows_v[r, pl.ds(c * 16, 16)] = jnp.zeros((16,), jnp.float32)
        return 0

    lax.fori_loop(0, _CHUNK * (DIM // 16), _zero_rows, 0)

    n_full = nz // _CHUNK
    rem = nz - n_full * _CHUNK
    for k in range(n_full):
        pltpu.sync_copy(rows_v, acc.at[pl.ds(base + k * _CHUNK, _CHUNK)])
    if rem:
        pltpu.sync_copy(rows_v.at[pl.ds(0, rem)],
                        acc.at[pl.ds(base + n_full * _CHUNK, rem)])

    @pl.when(sid == _NTILES - 1)
    def _():
        pltpu.sync_copy(rows_v.at[pl.ds(0, last_extra + 8)],
                        acc.at[pl.ds(nz * _NTILES, last_extra + 8)])

    plsc.subcore_barrier()

    pass_rows = gidx_v.shape[0]

    def run_dir(feat_hbm, gidx_hbm, sidx_hbm, out_hbm):
        def g_start(j, buf, sem):
            pltpu.async_copy(feat_hbm.at[gidx_v.at[j]], buf, sem)

        def g_wait(j, buf, sem):
            pltpu.make_async_copy(feat_hbm.at[gidx_v.at[j]], buf, sem).wait()

        def s_start(j, buf, sem):
            pltpu.async_copy(buf, acc.at[sidx_v.at[j]], sem, add=True)

        def s_wait(j, buf, sem):
            pltpu.make_async_copy(buf, acc.at[sidx_v.at[j]], sem).wait()

        def body(jj, _):
            j = jj * 2
            g_wait(j, rows_v, sem0)
            s_start(j, rows_v, sem0)
            g_wait(j + 1, rows_w, sem1)
            s_start(j + 1, rows_w, sem1)
            s_wait(j, rows_v, sem0)

            @pl.when(j + 2 < pass_rows)
            def _():
                g_start(j + 2, rows_v, sem0)

            s_wait(j + 1, rows_w, sem1)

            @pl.when(j + 3 < pass_rows)
            def _():
                g_start(j + 3, rows_w, sem1)

            return 0

        for h in range(rows_per_tile // pass_rows):
            off = sid * rows_per_tile + h * pass_rows
            pltpu.sync_copy(gidx_hbm.at[pl.ds(off, pass_rows)], gidx_v)
            pltpu.sync_copy(sidx_hbm.at[pl.ds(off, pass_rows)], sidx_v)
            g_start(0, rows_v, sem0)
            g_start(1, rows_w, sem1)
            lax.fori_loop(0, pass_rows // 2, body, 0)
        plsc.subcore_barrier()

        for k in range(n_full):
            pltpu.sync_copy(acc.at[pl.ds(base + k * _CHUNK, _CHUNK)],
                            out_hbm.at[pl.ds(base + k * _CHUNK, _CHUNK)])
        if rem:
            pltpu.sync_copy(acc.at[pl.ds(base + n_full * _CHUNK, rem)],
                            out_hbm.at[pl.ds(base + n_full * _CHUNK, rem)])

        if last_extra:
            @pl.when(sid == _NTILES - 1)
            def _():
                pltpu.sync_copy(acc.at[pl.ds(nz * _NTILES, last_extra)],
                                out_hbm.at[pl.ds(nz * _NTILES, last_extra)])

    @pl.when(cid == 0)
    def _():
        run_dir(featL, g0, s0, out0)

    @pl.when(cid == 1)
    def _():
        run_dir(featC, g1, s1, out1)


@functools.lru_cache(maxsize=None)
def _make_sc_scatter(n_nodes, idx_rows):
    rows_per_tile = idx_rows // _NTILES
    pass_rows = 8
    for cand in (40, 32, 24, 16):
        if rows_per_tile % cand == 0:
            pass_rows = cand
            break
    mesh = plsc.VectorSubcoreMesh(core_axis_name="c", subcore_axis_name="s")
    f32 = jnp.float32
    return pl.kernel(
        functools.partial(_sc_scatter_body, n_nodes, rows_per_tile),
        out_type=[jax.ShapeDtypeStruct((n_nodes, DIM), f32),
                  jax.ShapeDtypeStruct((n_nodes, DIM), f32)],
        mesh=mesh,
        scratch_types=[
            pltpu.VMEM_SHARED((n_nodes + 8, DIM), f32),
            pltpu.VMEM((pass_rows, _CHUNK), jnp.int32),
            pltpu.VMEM((pass_rows, _CHUNK), jnp.int32),
            pltpu.VMEM((_CHUNK, DIM), f32),
            pltpu.VMEM((_CHUNK, DIM), f32),
            pltpu.SemaphoreType.DMA,
            pltpu.SemaphoreType.DMA,
        ],
    )


_BLK = 1000


def _mlp_body(x_ref, w1_ref, b1_ref, w2_ref, b2_ref, o_ref):
    x = x_ref[...]
    h = jnp.maximum(
        jnp.dot(x, w1_ref[...], preferred_element_type=jnp.float32)
        + b1_ref[...], 0.0)
    o_ref[...] = (jnp.dot(h, w2_ref[...], preferred_element_type=jnp.float32)
                  + b2_ref[...])


def _mlp(x, w1, b1, w2, b2):
    n = x.shape[0]
    grid = n // _BLK
    return pl.pallas_call(
        _mlp_body,
        grid=(grid,),
        in_specs=[
            pl.BlockSpec((_BLK, DIM), lambda i: (i, 0)),
            pl.BlockSpec((DIM, DIM), lambda i: (0, 0)),
            pl.BlockSpec((1, DIM), lambda i: (0, 0)),
            pl.BlockSpec((DIM, DIM), lambda i: (0, 0)),
            pl.BlockSpec((1, DIM), lambda i: (0, 0)),
        ],
        out_specs=pl.BlockSpec((_BLK, DIM), lambda i: (i, 0)),
        out_shape=jax.ShapeDtypeStruct((n, DIM), jnp.float32),
    )(x, w1, b1.reshape(1, DIM), w2, b2.reshape(1, DIM))


def _gru_gates(gi, gh, h):
    r = jax.nn.sigmoid(gi[:, :DIM] + gh[:, :DIM])
    z = jax.nn.sigmoid(gi[:, DIM:2 * DIM] + gh[:, DIM:2 * DIM])
    n = jnp.tanh(gi[:, 2 * DIM:] + r * gh[:, 2 * DIM:])
    return (1.0 - z) * n + z * h


def _gru_c_body(x_ref, h_ref, wih_t, whh_t, bih, bhh, o_ref):
    x = x_ref[...]
    h = h_ref[...]
    gi = jnp.dot(x, wih_t[...], preferred_element_type=jnp.float32) + bih[...]
    gh = jnp.dot(h, whh_t[...], preferred_element_type=jnp.float32) + bhh[...]
    o_ref[...] = _gru_gates(gi, gh, h)


def _gru_c(x, h, wih_t, whh_t, bih, bhh):
    n = x.shape[0]
    grid = n // _BLK
    full = pl.BlockSpec((_BLK, DIM), lambda i: (i, 0))
    return pl.pallas_call(
        _gru_c_body,
        grid=(grid,),
        in_specs=[
            full, full,
            pl.BlockSpec((DIM, 3 * DIM), lambda i: (0, 0)),
            pl.BlockSpec((DIM, 3 * DIM), lambda i: (0, 0)),
            pl.BlockSpec((1, 3 * DIM), lambda i: (0, 0)),
            pl.BlockSpec((1, 3 * DIM), lambda i: (0, 0)),
        ],
        out_specs=full,
        out_shape=jax.ShapeDtypeStruct((n, DIM), jnp.float32),
    )(x, h, wih_t, whh_t, bih.reshape(1, -1), bhh.reshape(1, -1))


def _gru_l_body(a_ref, h_ref, wa_t, wb_t, whh_t, bih, bhh, o_ref):
    a = a_ref[...]
    h = h_ref[...]
    up = jnp.roll(h, -1, axis=0)
    down = jnp.roll(h, 1, axis=0)
    parity = lax.rem(lax.broadcasted_iota(jnp.int32, h.shape, 0), 2)
    swapped = jnp.where(parity == 0, up, down)
    gi = (jnp.dot(a, wa_t[...], preferred_element_type=jnp.float32)
          + jnp.dot(swapped, wb_t[...], preferred_element_type=jnp.float32)
          + bih[...])
    gh = jnp.dot(h, whh_t[...], preferred_element_type=jnp.float32) + bhh[...]
    o_ref[...] = _gru_gates(gi, gh, h)


def _gru_l(a, h, wa_t, wb_t, whh_t, bih, bhh):
    n = a.shape[0]
    grid = n // _BLK
    full = pl.BlockSpec((_BLK, DIM), lambda i: (i, 0))
    w = pl.BlockSpec((DIM, 3 * DIM), lambda i: (0, 0))
    return pl.pallas_call(
        _gru_l_body,
        grid=(grid,),
        in_specs=[
            full, full, w, w, w,
            pl.BlockSpec((1, 3 * DIM), lambda i: (0, 0)),
            pl.BlockSpec((1, 3 * DIM), lambda i: (0, 0)),
        ],
        out_specs=full,
        out_shape=jax.ShapeDtypeStruct((n, DIM), jnp.float32),
    )(a, h, wa_t, wb_t, whh_t, bih.reshape(1, -1), bhh.reshape(1, -1))


def kernel(l_size, c_size, l_edge_index, c_edge_index, l_emb, c_emb,
           W1_l2c, b1_l2c, W2_l2c, b2_l2c,
           W1_c2l, b1_c2l, W2_c2l, b2_c2l,
           Wih_c, Whh_c, bih_c, bhh_c,
           Wih_l, Whh_l, bih_l, bhh_l):
    ls = l_emb.shape[0]
    cs = c_emb.shape[0]
    n_nodes = ls
    e = l_edge_index.shape[0]

    per_tile = -(-e // (_NTILES * _CHUNK))
    per_tile = -(-per_tile // 8) * 8
    e_pad = per_tile * _NTILES * _CHUNK
    trash = jnp.int32(n_nodes)
    ei_l = l_edge_index.astype(jnp.int32)
    ei_c = c_edge_index.astype(jnp.int32)
    pad0 = jnp.zeros((e_pad - e,), jnp.int32)
    padt = jnp.full((e_pad - e,), trash, jnp.int32)
    rows = e_pad // _CHUNK
    g0 = jnp.concatenate([ei_l, pad0]).reshape(rows, _CHUNK)
    s0 = jnp.concatenate([ei_c, padt]).reshape(rows, _CHUNK)
    g1 = jnp.concatenate([ei_c, pad0]).reshape(rows, _CHUNK)
    s1 = jnp.concatenate([ei_l, padt]).reshape(rows, _CHUNK)

    sc_scatter = _make_sc_scatter(n_nodes, rows)

    wih_c_t = Wih_c.T
    whh_c_t = Whh_c.T
    wih_l_a = Wih_l[:, :DIM].T
    wih_l_b = Wih_l[:, DIM:].T
    whh_l_t = Whh_l.T

    l_embs = [l_emb]
    c_embs = [c_emb]
    for _ in range(N_ITER):
        featL = _mlp(l_emb, W1_l2c, b1_l2c, W2_l2c, b2_l2c)
        featC = _mlp(c_emb, W1_c2l, b1_c2l, W2_c2l, b2_c2l)
        l2c_aggr, c2l_aggr = sc_scatter(featL, featC, g0, s0, g1, s1)
        c_emb = _gru_c(l2c_aggr, c_emb, wih_c_t, whh_c_t, bih_c, bhh_c)
        l_emb = _gru_l(c2l_aggr, l_emb, wih_l_a, wih_l_b, whh_l_t,
                       bih_l, bhh_l)
        l_embs.append(l_emb)
        c_embs.append(c_emb)
    return (jnp.stack(l_embs), jnp.stack(c_embs))
